# Initial kernel scaffold; baseline (speedup 1.0000x reference)
import jax
import jax.numpy as jnp
from jax.experimental import pallas as pl


def kernel(x, adj_t, y, W_in, b_in, W_h, b_h, W_out, b_out):
    raise NotImplementedError("write your pallas kernel here")


# speedup vs baseline: 7.1654x; 7.1654x over previous
import functools

import jax
import jax.numpy as jnp
from jax import lax
from jax.experimental import pallas as pl
from jax.experimental.pallas import tpu as pltpu
from jax.experimental.pallas import tpu_sc as plsc

N = 10000
E = 320000
D = 128
H = 128
C = 64

NP = 10240
EP = 327680
NCORES = 2
NSUB = 16
NTILES = NCORES * NSUB
KCH = 128
NCHUNK = EP // (NTILES * KCH)
RPT = NP // NSUB
BN = 512
GRID = NP // BN

@functools.cache
def _mesh():
  return plsc.VectorSubcoreMesh(
      core_axis_name="c", subcore_axis_name="s",
      num_cores=NCORES, num_subcores=NSUB)


@functools.cache
def _make_segsum(width):
  @functools.partial(
      pl.kernel,
      out_type=jax.ShapeDtypeStruct((NCORES, NP, width), jnp.float32),
      mesh=_mesh(),
      scratch_types=[
          pltpu.VMEM((NCHUNK, KCH), jnp.int32),
          pltpu.VMEM((NCHUNK, KCH), jnp.int32),
          pltpu.VMEM((KCH, width), jnp.float32),
          pltpu.VMEM_SHARED((NP, width), jnp.float32),
          pltpu.SemaphoreType.DMA,
      ],
  )
  def seg(g_hbm, src_hbm, dst_hbm, out_hbm, s_v, d_v, rows_v, acc_sh, sem):
    cid = lax.axis_index("c")
    sid = lax.axis_index("s")
    tid = sid * NCORES + cid
    pltpu.sync_copy(src_hbm.at[pl.ds(tid * NCHUNK, NCHUNK)], s_v)
    pltpu.sync_copy(dst_hbm.at[pl.ds(tid * NCHUNK, NCHUNK)], d_v)

    def zrow(r, carry):
      def zcol(c2, carry2):
        rows_v[r, pl.ds(c2 * 16, 16)] = jnp.zeros((16,), jnp.float32)
        return carry2
      return lax.fori_loop(0, width // 16, zcol, carry)
    lax.fori_loop(0, KCH, zrow, 0)

    def zacc(b, carry):
      pltpu.sync_copy(rows_v, acc_sh.at[pl.ds(sid * RPT + b * KCH, KCH)])
      return carry
    lax.fori_loop(0, RPT // KCH, zacc, 0)
    plsc.subcore_barrier()

    def body(j, carry):
      pltpu.async_copy(g_hbm.at[s_v.at[j]], rows_v, sem).wait()
      pltpu.sync_copy(rows_v, acc_sh.at[d_v.at[j]], add=True)
      return carry
    lax.fori_loop(0, NCHUNK, body, 0)
    plsc.subcore_barrier()

    def wout(b, carry):
      pltpu.sync_copy(acc_sh.at[pl.ds(sid * RPT + b * KCH, KCH)],
                      out_hbm.at[cid, pl.ds(sid * RPT + b * KCH, KCH)])
      return carry
    lax.fori_loop(0, RPT // KCH, wout, 0)

  return seg


@functools.cache
def _make_deg_kernel():

  @functools.partial(
      pl.kernel,
      out_type=jax.ShapeDtypeStruct((NCORES, NP, 16), jnp.float32),
      mesh=_mesh(),
      scratch_types=[
          pltpu.VMEM((NCHUNK, KCH), jnp.int32),
          pltpu.VMEM((KCH, 16), jnp.float32),
          pltpu.VMEM_SHARED((NP, 16), jnp.float32),
      ],
  )
  def _deg_kernel(dst_hbm, out_hbm, d_v, ones_v, acc_sh):
    cid = lax.axis_index("c")
    sid = lax.axis_index("s")
    tid = sid * NCORES + cid
    pltpu.sync_copy(dst_hbm.at[pl.ds(tid * NCHUNK, NCHUNK)], d_v)

    def zrow(r, carry):
      ones_v[r, pl.ds(0, 16)] = jnp.zeros((16,), jnp.float32)
      return carry
    lax.fori_loop(0, KCH, zrow, 0)

    def zacc(b, carry):
      pltpu.sync_copy(ones_v, acc_sh.at[pl.ds(sid * RPT + b * KCH, KCH)])
      return carry
    lax.fori_loop(0, RPT // KCH, zacc, 0)

    def orow(r, carry):
      ones_v[r, pl.ds(0, 16)] = jnp.ones((16,), jnp.float32)
      return carry
    lax.fori_loop(0, KCH, orow, 0)
    plsc.subcore_barrier()

    def body(j, carry):
      pltpu.sync_copy(ones_v, acc_sh.at[d_v.at[j]], add=True)
      return carry
    lax.fori_loop(0, NCHUNK, body, 0)
    plsc.subcore_barrier()

    def wout(b, carry):
      pltpu.sync_copy(acc_sh.at[pl.ds(sid * RPT + b * KCH, KCH)],
                      out_hbm.at[cid, pl.ds(sid * RPT + b * KCH, KCH)])
      return carry
    lax.fori_loop(0, RPT // KCH, wout, 0)

  return _deg_kernel


def _invs_block(pd_ref, i):
  deg = pd_ref[0, :, 0:1] + pd_ref[1, :, 0:1] + 1.0
  rows = i * BN + lax.broadcasted_iota(jnp.int32, (BN, 1), 0)
  return jnp.where(rows < N, lax.rsqrt(deg), 0.0)


def _rowmask(i):
  rows = i * BN + lax.broadcasted_iota(jnp.int32, (BN, 1), 0)
  return rows < N


def _a1_body(x_ref, w_ref, pd_ref, g_ref):
  i = pl.program_id(0)
  invs = _invs_block(pd_ref, i)
  h = jnp.dot(x_ref[...], w_ref[...], preferred_element_type=jnp.float32)
  g_ref[...] = h * invs


def _b_body(p_ref, g_ref, pd_ref, b_ref, conv_ref, sums_ref):
  i = pl.program_id(0)
  invs = _invs_block(pd_ref, i)
  t = (p_ref[0] + p_ref[1] + g_ref[...]) * invs + b_ref[...]
  conv_ref[...] = t
  tm = jnp.where(_rowmask(i), t, 0.0)
  s1 = jnp.sum(tm)
  s2 = jnp.sum(tm * tm)
  lane = lax.broadcasted_iota(jnp.int32, (1, 128), 1)
  v = jnp.where(lane == 0, s1, 0.0) + jnp.where(lane == 1, s2, 0.0)

  @pl.when(i == 0)
  def _():
    sums_ref[...] = v

  @pl.when(i > 0)
  def _():
    sums_ref[...] = sums_ref[...] + v


def _norm_scalars(sums_ref):
  cnt = float(N * H)
  s1 = sums_ref[0, 0]
  s2 = sums_ref[0, 1]
  m = s1 / cnt
  var = (s2 - s1 * s1 / cnt) / (cnt - 1.0)
  return m, jnp.sqrt(var) + 1e-6


def _a2_body(conv_ref, sums_ref, w_ref, pd_ref, xin_ref, g_ref):
  i = pl.program_id(0)
  m, std = _norm_scalars(sums_ref)
  invs = _invs_block(pd_ref, i)
  xin = jnp.maximum((conv_ref[...] - m) / std, 0.0)
  xin = jnp.where(_rowmask(i), xin, 0.0)
  xin_ref[...] = xin
  g_ref[...] = jnp.dot(xin, w_ref[...], preferred_element_type=jnp.float32) * invs


def _a3_body(conv_ref, sums_ref, xin2_ref, w_ref, pd_ref, g_ref):
  i = pl.program_id(0)
  m, std = _norm_scalars(sums_ref)
  invs = _invs_block(pd_ref, i)
  xin = jnp.maximum((conv_ref[...] - m) / std, 0.0)
  xin = jnp.where(_rowmask(i), xin, 0.0) + xin2_ref[...]
  g_ref[...] = jnp.dot(xin, w_ref[...], preferred_element_type=jnp.float32) * invs


def _b3_body(p_ref, g_ref, pd_ref, b_ref, out_ref):
  i = pl.program_id(0)
  invs = _invs_block(pd_ref, i)
  t = (p_ref[0] + p_ref[1] + g_ref[...])[:, :C] * invs + b_ref[...]
  mx = jnp.max(t, axis=1, keepdims=True)
  lse = jnp.log(jnp.sum(jnp.exp(t - mx), axis=1, keepdims=True)) + mx
  out_ref[...] = t - lse


def _row_spec(width):
  return pl.BlockSpec((BN, width), lambda i: (i, 0))


_PD_SPEC = pl.BlockSpec((NCORES, BN, 16), lambda i: (0, i, 0))
_SUMS_SPEC = pl.BlockSpec((1, 128), lambda i: (0, 0))


def _full_spec(shape):
  return pl.BlockSpec(shape, lambda i: tuple(0 for _ in shape))


def _parts_spec(width):
  return pl.BlockSpec((NCORES, BN, width), lambda i: (0, i, 0))


_a1 = pl.pallas_call(
    _a1_body,
    grid=(GRID,),
    in_specs=[_row_spec(D), _full_spec((D, H)), _PD_SPEC],
    out_specs=_row_spec(H),
    out_shape=jax.ShapeDtypeStruct((NP, H), jnp.float32),
)

_b128 = pl.pallas_call(
    _b_body,
    grid=(GRID,),
    in_specs=[_parts_spec(H), _row_spec(H), _PD_SPEC, _full_spec((H,))],
    out_specs=[_row_spec(H), _SUMS_SPEC],
    out_shape=[
        jax.ShapeDtypeStruct((NP, H), jnp.float32),
        jax.ShapeDtypeStruct((1, 128), jnp.float32),
    ],
)

_a2 = pl.pallas_call(
    _a2_body,
    grid=(GRID,),
    in_specs=[_row_spec(H), _SUMS_SPEC, _full_spec((H, H)), _PD_SPEC],
    out_specs=[_row_spec(H), _row_spec(H)],
    out_shape=[
        jax.ShapeDtypeStruct((NP, H), jnp.float32),
        jax.ShapeDtypeStruct((NP, H), jnp.float32),
    ],
)

_a3 = pl.pallas_call(
    _a3_body,
    grid=(GRID,),
    in_specs=[_row_spec(H), _SUMS_SPEC, _row_spec(H), _full_spec((H, H)),
              _PD_SPEC],
    out_specs=_row_spec(H),
    out_shape=jax.ShapeDtypeStruct((NP, H), jnp.float32),
)

_b3 = pl.pallas_call(
    _b3_body,
    grid=(GRID,),
    in_specs=[_parts_spec(H), _row_spec(H), _PD_SPEC, _full_spec((C,))],
    out_specs=_row_spec(C),
    out_shape=jax.ShapeDtypeStruct((NP, C), jnp.float32),
)


def kernel(x, adj_t, y, W_in, b_in, W_h, b_h, W_out, b_out):
  src = adj_t[0]
  dst = adj_t[1]
  padv = jnp.full((EP - E,), NP - 1, dtype=jnp.int32)
  srcp = jnp.concatenate([src, padv]).reshape(EP // KCH, KCH)
  dstp = jnp.concatenate([dst, padv]).reshape(EP // KCH, KCH)
  xp = jnp.zeros((NP, D), jnp.float32).at[:N].set(x)

  W_out_p = jnp.zeros((H, H), jnp.float32).at[:, :C].set(W_out)

  segsum128 = _make_segsum(H)
  pd = _make_deg_kernel()(dstp)
  g1 = _a1(xp, W_in, pd)
  p1 = segsum128(g1, srcp, dstp)
  conv1, sums1 = _b128(p1, g1, pd, b_in)
  xin2, g2 = _a2(conv1, sums1, W_h, pd)
  p2 = segsum128(g2, srcp, dstp)
  conv2, sums2 = _b128(p2, g2, pd, b_h)
  g3 = _a3(conv2, sums2, xin2, W_out_p, pd)
  p3 = segsum128(g3, srcp, dstp)
  out = _b3(p3, g3, pd, b_out)
  return out[:N]
